# baseline (speedup 1.0000x reference)
import functools

import jax
import jax.numpy as jnp
from jax import lax
from jax.experimental import pallas as pl
from jax.experimental.pallas import tpu as pltpu
from jax.experimental.pallas import tpu_sc as plsc

NUM_EXPERTS = 8
TOP_K = 2
HIDDEN = 1024
INTER = 2816
SEQ = 2048

BLK = 256
NASSIGN = SEQ * TOP_K
NB = NASSIGN // BLK + NUM_EXPERTS
NPAD = NB * BLK


IB = 1408
NJ = INTER // IB


_SC_NC = 2
_SC_NS = 16
_SC_NW = _SC_NC * _SC_NS
_BPW = NPAD // _SC_NW
_CH = 64


def _sc_gather_body(x_hbm, idx_hbm, out_hbm, idx_v, rows_v, sem):
    wid = lax.axis_index("s") * _SC_NC + lax.axis_index("c")
    base = wid * _BPW
    pltpu.sync_copy(idx_hbm.at[pl.ds(base, _BPW)], idx_v)
    for c in range(_BPW // _CH):
        cp = pltpu.async_copy(
            x_hbm.at[idx_v.at[pl.ds(c * _CH, _CH)]], rows_v, sem)
        cp.wait()
        pltpu.sync_copy(rows_v, out_hbm.at[pl.ds(base + c * _CH, _CH)])


def _sc_gather(x2d, src_token):
    mesh = plsc.VectorSubcoreMesh(core_axis_name="c", subcore_axis_name="s")
    return pl.kernel(
        _sc_gather_body,
        mesh=mesh,
        out_type=jax.ShapeDtypeStruct((NPAD, HIDDEN), jnp.float32),
        scratch_types=[
            pltpu.VMEM((_BPW,), jnp.int32),
            pltpu.VMEM((_CH, HIDDEN), jnp.float32),
            pltpu.SemaphoreType.DMA,
        ],
    )(x2d, src_token)


def _moe_mm_kernel(be_ref, nact_ref, xp_ref, w1_ref, w3_ref, w2_ref, wt_ref,
                   y_ref):
    j = pl.program_id(0)
    b = pl.program_id(1)
    nact = nact_ref[0]

    @pl.when(b < nact)
    def _active():
        x = xp_ref[...]
        g = jnp.dot(x, w1_ref[0], preferred_element_type=jnp.float32)
        u = jnp.dot(x, w3_ref[0], preferred_element_type=jnp.float32)
        h = (g * jax.nn.sigmoid(g)) * u
        y = jnp.dot(h, w2_ref[0], preferred_element_type=jnp.float32)
        y_ref[0] = y * wt_ref[...]

    @pl.when(b >= nact)
    def _inactive():
        y_ref[...] = jnp.zeros_like(y_ref)


def _grouped_mlp(x_padded, w1, w3, w2, wt_padded, block_expert, nact):
    grid_spec = pltpu.PrefetchScalarGridSpec(
        num_scalar_prefetch=2,
        grid=(NJ, NB),
        in_specs=[
            pl.BlockSpec((BLK, HIDDEN), lambda j, b, be, na: (b, 0)),
            pl.BlockSpec((1, HIDDEN, IB), lambda j, b, be, na: (be[b], 0, j)),
            pl.BlockSpec((1, HIDDEN, IB), lambda j, b, be, na: (be[b], 0, j)),
            pl.BlockSpec((1, IB, HIDDEN), lambda j, b, be, na: (be[b], j, 0)),
            pl.BlockSpec((BLK, 1), lambda j, b, be, na: (b, 0)),
        ],
        out_specs=pl.BlockSpec((1, BLK, HIDDEN), lambda j, b, be, na: (j, b, 0)),
    )
    return pl.pallas_call(
        _moe_mm_kernel,
        grid_spec=grid_spec,
        out_shape=jax.ShapeDtypeStruct((NJ, NPAD, HIDDEN), jnp.float32),
    )(block_expert, nact, x_padded, w1, w3, w2, wt_padded)


def kernel(x, gate, experts_w1, experts_w2, experts_w3):
    x2d = x.reshape(SEQ, HIDDEN)

    logits = jnp.matmul(x2d, gate)
    top_w, top_ids = jax.lax.top_k(logits, TOP_K)
    top_w = jax.nn.softmax(top_w, axis=-1)

    e_flat = top_ids.reshape(-1)
    w_flat = top_w.reshape(-1).astype(jnp.float32)
    t_flat = jnp.arange(NASSIGN, dtype=jnp.int32) // TOP_K
    order = jnp.argsort(e_flat, stable=True)
    e_sorted = e_flat[order]
    sorted_t = t_flat[order]
    sorted_w = w_flat[order]
    counts = jnp.bincount(e_flat, length=NUM_EXPERTS)
    start = jnp.concatenate([jnp.zeros(1, counts.dtype), jnp.cumsum(counts)[:-1]])
    nblocks_e = (counts + BLK - 1) // BLK
    padded_start = jnp.concatenate(
        [jnp.zeros(1, counts.dtype), jnp.cumsum(nblocks_e * BLK)[:-1]])
    rank = jnp.arange(NASSIGN, dtype=counts.dtype) - start[e_sorted]
    slot = (padded_start[e_sorted] + rank).astype(jnp.int32)

    blocks_cum = jnp.cumsum(nblocks_e)
    nact = blocks_cum[-1].astype(jnp.int32)
    bclamp = jnp.minimum(jnp.arange(NB), nact - 1)
    block_expert = jnp.searchsorted(blocks_cum, bclamp, side="right").astype(jnp.int32)

    src_token = jnp.zeros(NPAD, jnp.int32).at[slot].set(sorted_t)
    wt_padded = jnp.zeros(NPAD, jnp.float32).at[slot].set(sorted_w)

    x_padded = _sc_gather(x2d, src_token)

    y_parts = _grouped_mlp(x_padded, experts_w1, experts_w3, experts_w2,
                           wt_padded.reshape(NPAD, 1), block_expert,
                           nact.reshape(1))
    yp = y_parts.reshape(NJ * NPAD, HIDDEN)

    slot_tk = jnp.zeros(NASSIGN, jnp.int32).at[order].set(slot).reshape(SEQ, TOP_K)
    s0, s1 = slot_tk[:, 0], slot_tk[:, 1]
    out = (yp.at[s0].get(mode="clip") + yp.at[s0 + NPAD].get(mode="clip") +
           yp.at[s1].get(mode="clip") + yp.at[s1 + NPAD].get(mode="clip"))
    return out.reshape(x.shape)


# speedup vs baseline: 1.2700x; 1.2700x over previous
import functools

import jax
import jax.numpy as jnp
from jax import lax
from jax.experimental import pallas as pl
from jax.experimental.pallas import tpu as pltpu
from jax.experimental.pallas import tpu_sc as plsc

NUM_EXPERTS = 8
TOP_K = 2
HIDDEN = 1024
INTER = 2816
SEQ = 2048

BLK = 256
NASSIGN = SEQ * TOP_K
NB = NASSIGN // BLK + NUM_EXPERTS
NPAD = NB * BLK


IB = 1408
NJ = INTER // IB


_SC_NC = 2
_SC_NS = 16
_SC_NW = _SC_NC * _SC_NS
_BPW = NPAD // _SC_NW
_CH = 64


def _sc_gather_body(x_hbm, idx_hbm, out_hbm, idx_v, rows_v, sem):
    wid = lax.axis_index("s") * _SC_NC + lax.axis_index("c")
    base = wid * _BPW
    pltpu.sync_copy(idx_hbm.at[pl.ds(base, _BPW)], idx_v)
    for c in range(_BPW // _CH):
        cp = pltpu.async_copy(
            x_hbm.at[idx_v.at[pl.ds(c * _CH, _CH)]], rows_v, sem)
        cp.wait()
        pltpu.sync_copy(rows_v, out_hbm.at[pl.ds(base + c * _CH, _CH)])


def _sc_gather(x2d, src_token):
    mesh = plsc.VectorSubcoreMesh(core_axis_name="c", subcore_axis_name="s")
    return pl.kernel(
        _sc_gather_body,
        mesh=mesh,
        out_type=jax.ShapeDtypeStruct((NPAD, HIDDEN), jnp.float32),
        scratch_types=[
            pltpu.VMEM((_BPW,), jnp.int32),
            pltpu.VMEM((_CH, HIDDEN), jnp.float32),
            pltpu.SemaphoreType.DMA,
        ],
    )(x2d, src_token)


def _moe_mm_kernel(be_ref, nact_ref, xp_ref, w1_ref, w3_ref, w2_ref, wt_ref,
                   y_ref):
    j = pl.program_id(0)
    b = pl.program_id(1)
    nact = nact_ref[0]

    @pl.when(b < nact)
    def _active():
        x = xp_ref[...]
        g = jnp.dot(x, w1_ref[0], preferred_element_type=jnp.float32)
        u = jnp.dot(x, w3_ref[0], preferred_element_type=jnp.float32)
        h = (g * jax.nn.sigmoid(g)) * u
        y = jnp.dot(h, w2_ref[0], preferred_element_type=jnp.float32)
        y_ref[0] = y * wt_ref[...]

    @pl.when(b >= nact)
    def _inactive():
        y_ref[...] = jnp.zeros_like(y_ref)


def _grouped_mlp(x_padded, w1, w3, w2, wt_padded, block_expert, nact):
    grid_spec = pltpu.PrefetchScalarGridSpec(
        num_scalar_prefetch=2,
        grid=(NJ, NB),
        in_specs=[
            pl.BlockSpec((BLK, HIDDEN), lambda j, b, be, na: (b, 0)),
            pl.BlockSpec((1, HIDDEN, IB), lambda j, b, be, na: (be[b], 0, j)),
            pl.BlockSpec((1, HIDDEN, IB), lambda j, b, be, na: (be[b], 0, j)),
            pl.BlockSpec((1, IB, HIDDEN), lambda j, b, be, na: (be[b], j, 0)),
            pl.BlockSpec((BLK, 1), lambda j, b, be, na: (b, 0)),
        ],
        out_specs=pl.BlockSpec((1, BLK, HIDDEN), lambda j, b, be, na: (j, b, 0)),
    )
    return pl.pallas_call(
        _moe_mm_kernel,
        grid_spec=grid_spec,
        out_shape=jax.ShapeDtypeStruct((NJ, NPAD, HIDDEN), jnp.float32),
    )(block_expert, nact, x_padded, w1, w3, w2, wt_padded)


def kernel(x, gate, experts_w1, experts_w2, experts_w3):
    x2d = x.reshape(SEQ, HIDDEN)

    logits = jnp.matmul(x2d, gate)
    top_w, top_ids = jax.lax.top_k(logits, TOP_K)
    top_w = jax.nn.softmax(top_w, axis=-1)

    e_flat = top_ids.reshape(-1)
    w_flat = top_w.reshape(-1).astype(jnp.float32)
    t_flat = jnp.arange(NASSIGN, dtype=jnp.int32) // TOP_K
    onehot = (e_flat[:, None] == jnp.arange(NUM_EXPERTS)[None, :]).astype(jnp.int32)
    csum = jnp.cumsum(onehot, axis=0)
    counts = csum[-1]
    rank = jnp.sum(csum * onehot, axis=1) - 1
    nblocks_e = (counts + BLK - 1) // BLK
    padded_start = jnp.concatenate(
        [jnp.zeros(1, counts.dtype), jnp.cumsum(nblocks_e * BLK)[:-1]])
    slot = (jnp.sum(onehot * padded_start[None, :], axis=1) + rank).astype(jnp.int32)

    blocks_cum = jnp.cumsum(nblocks_e)
    nact = blocks_cum[-1].astype(jnp.int32)
    bclamp = jnp.minimum(jnp.arange(NB), nact - 1)
    block_expert = jnp.searchsorted(blocks_cum, bclamp, side="right").astype(jnp.int32)

    src_token = jnp.zeros(NPAD, jnp.int32).at[slot].set(t_flat)
    wt_padded = jnp.zeros(NPAD, jnp.float32).at[slot].set(w_flat)

    x_padded = x2d.at[src_token].get(mode="clip")

    y_parts = _grouped_mlp(x_padded, experts_w1, experts_w3, experts_w2,
                           wt_padded.reshape(NPAD, 1), block_expert,
                           nact.reshape(1))
    yp = y_parts.reshape(NJ * NPAD, HIDDEN)

    slot_tk = slot.reshape(SEQ, TOP_K)
    s0, s1 = slot_tk[:, 0], slot_tk[:, 1]
    out = (yp.at[s0].get(mode="clip") + yp.at[s0 + NPAD].get(mode="clip") +
           yp.at[s1].get(mode="clip") + yp.at[s1 + NPAD].get(mode="clip"))
    return out.reshape(x.shape)
